# baseline (speedup 1.0000x reference)
import functools

import jax
import jax.numpy as jnp
from jax import lax
from jax.experimental import pallas as pl
from jax.experimental.pallas import tpu as pltpu
from jax.experimental.pallas import tpu_sc as plsc

N_VOX = 500000
CH = 64
B = 8
VOX = 0.0625
BX = 6.0
BY = 6.0
BZ = 3.0

NC = 2
NS = 16
NW = NC * NS
LANES = 16
CHUNK = 4000
PER_W = 16000
N_PAD = NW * PER_W
R_BLK = 16384


def _col(ref, i, j):
    return ref[:, 4 * i + j:4 * i + j + 1]


def _transform_body(inv_ref, hist_ref, out_ref):
    inv_b = inv_ref[...].astype(jnp.bfloat16).astype(jnp.float32)
    hist_b = hist_ref[...].astype(jnp.bfloat16).astype(jnp.float32)
    binv = [[inv_b[:, 4 * i + j:4 * i + j + 1] for j in range(4)]
            for i in range(4)]
    h = [[hist_b[:, 4 * i + j:4 * i + j + 1] for j in range(4)]
         for i in range(4)]
    cols = []
    for i in range(4):
        for k in range(4):
            cols.append(sum(binv[i][j] * h[j][k] for j in range(4)))
    out_ref[...] = jnp.concatenate(cols, axis=1)


def _compute_transform(inv_flat, hist_flat):
    return pl.pallas_call(
        _transform_body,
        out_shape=jax.ShapeDtypeStruct((B, 16), jnp.float32),
    )(inv_flat, hist_flat)


def _sc_body(cx_h, cy_h, cz_h, bi_h, sdf_h, occ_h, t_h,
             hx_h, hy_h, hz_h, mf_h, nb_h, ps_h, po_h,
             cx_v, cy_v, cz_v, bi_v, sdf_v, occ_v,
             hx_v, hy_v, hz_v, mf_v, nb_v, ps_v, po_v, t_v):
    wid = lax.axis_index("s") * NC + lax.axis_index("c")
    pltpu.sync_copy(t_h, t_v)
    for c in range(PER_W // CHUNK):
        base = wid * PER_W + c * CHUNK
        pltpu.sync_copy(cx_h.at[pl.ds(base, CHUNK)], cx_v)
        pltpu.sync_copy(cy_h.at[pl.ds(base, CHUNK)], cy_v)
        pltpu.sync_copy(cz_h.at[pl.ds(base, CHUNK)], cz_v)
        pltpu.sync_copy(bi_h.at[pl.ds(base, CHUNK)], bi_v)
        pltpu.sync_copy(sdf_h.at[pl.ds(base, CHUNK)], sdf_v)
        pltpu.sync_copy(occ_h.at[pl.ds(base, CHUNK)], occ_v)

        def body(i, carry):
            s = i * LANES
            bi = bi_v[pl.ds(s, LANES)]
            nb = lax.rem(bi, B)
            nb16 = nb * 16
            t = [plsc.load_gather(t_v, [nb16 + k]) for k in range(12)]
            cx = cx_v[pl.ds(s, LANES)]
            cy = cy_v[pl.ds(s, LANES)]
            cz = cz_v[pl.ds(s, LANES)]
            hx = cx * t[0] + cy * t[1] + cz * t[2] + t[3]
            hy = cx * t[4] + cy * t[5] + cz * t[6] + t[7]
            hz = cx * t[8] + cy * t[9] + cz * t[10] + t[11]
            m = ((hx >= 0.0) & (hx < BX) & (hy >= 0.0) & (hy < BY)
                 & (hz >= 0.0) & (hz < BZ))
            zero = jnp.zeros((LANES,), jnp.float32)
            hx_v[pl.ds(s, LANES)] = hx
            hy_v[pl.ds(s, LANES)] = hy
            hz_v[pl.ds(s, LANES)] = hz
            mf_v[pl.ds(s, LANES)] = jnp.where(m, 1.0, zero)
            nb_v[pl.ds(s, LANES)] = nb
            ps_v[pl.ds(s, LANES)] = jnp.where(m, sdf_v[pl.ds(s, LANES)], zero)
            po_v[pl.ds(s, LANES)] = jnp.where(m, occ_v[pl.ds(s, LANES)], zero)
            return carry

        lax.fori_loop(0, CHUNK // LANES, body, 0)
        pltpu.sync_copy(hx_v, hx_h.at[pl.ds(base, CHUNK)])
        pltpu.sync_copy(hy_v, hy_h.at[pl.ds(base, CHUNK)])
        pltpu.sync_copy(hz_v, hz_h.at[pl.ds(base, CHUNK)])
        pltpu.sync_copy(mf_v, mf_h.at[pl.ds(base, CHUNK)])
        pltpu.sync_copy(nb_v, nb_h.at[pl.ds(base, CHUNK)])
        pltpu.sync_copy(ps_v, ps_h.at[pl.ds(base, CHUNK)])
        pltpu.sync_copy(po_v, po_h.at[pl.ds(base, CHUNK)])


def _sc_rows(cx, cy, cz, bi, sdf_c, occ_c, t_flat):
    f32 = jnp.float32
    i32 = jnp.int32
    vmem_f = pltpu.VMEM((CHUNK,), f32)
    vmem_i = pltpu.VMEM((CHUNK,), i32)
    mesh = plsc.VectorSubcoreMesh(core_axis_name="c", subcore_axis_name="s")
    fn = functools.partial(
        pl.kernel,
        mesh=mesh,
        compiler_params=pltpu.CompilerParams(needs_layout_passes=False),
        out_type=[
            jax.ShapeDtypeStruct((N_PAD,), f32),
            jax.ShapeDtypeStruct((N_PAD,), f32),
            jax.ShapeDtypeStruct((N_PAD,), f32),
            jax.ShapeDtypeStruct((N_PAD,), f32),
            jax.ShapeDtypeStruct((N_PAD,), i32),
            jax.ShapeDtypeStruct((N_PAD,), f32),
            jax.ShapeDtypeStruct((N_PAD,), f32),
        ],
        scratch_types=[
            vmem_f, vmem_f, vmem_f, vmem_i, vmem_f, vmem_f,
            vmem_f, vmem_f, vmem_f, vmem_f, vmem_i, vmem_f, vmem_f,
            pltpu.VMEM((B * 16,), f32),
        ],
    )(_sc_body)
    return fn(cx, cy, cz, bi, sdf_c, occ_c, t_flat)


VR = 8192
N_VIEW = N_VOX * CH // 128


def _feat_body(f_ref, m_ref, o_ref):
    m = m_ref[...]
    mrep = jnp.broadcast_to(m[:, None, :], (128, 64, 128)).reshape(VR, 128)
    row = lax.broadcasted_iota(jnp.int32, (VR, 128), 0)
    lane = lax.broadcasted_iota(jnp.int32, (VR, 128), 1)
    two_p = 2 * (row % 64)
    sel_e = (lane == two_p).astype(jnp.float32)
    sel_o = (lane == two_p + 1).astype(jnp.float32)
    ones = jnp.ones((128, 1), jnp.float32)
    mcol_e = jnp.dot(mrep * sel_e, ones)
    mcol_o = jnp.dot(mrep * sel_o, ones)
    lanes = lax.broadcasted_iota(jnp.int32, (VR, 128), 1)
    mexp = jnp.where(lanes < 64, mcol_e, mcol_o)
    o_ref[...] = f_ref[...] * mexp


def _mask_features(fview, mask_rows):
    grid = pl.cdiv(N_VIEW, VR)
    return pl.pallas_call(
        _feat_body,
        grid=(grid,),
        in_specs=[
            pl.BlockSpec((VR, 128), lambda i: (i, 0)),
            pl.BlockSpec((VR // 64, 128), lambda i: (i, 0)),
        ],
        out_specs=pl.BlockSpec((VR, 128), lambda i: (i, 0)),
        out_shape=jax.ShapeDtypeStruct((N_VIEW, 128), jnp.float32),
    )(fview, mask_rows)


def kernel(coords, batch_inds, features, sdf, occupancy,
           historical_pose, current_pose):
    n = coords.shape[0]
    pad = N_PAD - n

    inv_current = jnp.linalg.inv(current_pose)
    t_flat = _compute_transform(
        inv_current.reshape(B, 16), historical_pose.reshape(B, 16))

    cx = jnp.pad(coords[:, 0], (0, pad))
    cy = jnp.pad(coords[:, 1], (0, pad))
    cz = jnp.pad(coords[:, 2], (0, pad))
    bi = jnp.pad(batch_inds, (0, pad))
    sdf_c = jnp.pad(sdf[:, 0], (0, pad))
    occ_c = jnp.pad(occupancy[:, 0], (0, pad))

    hx, hy, hz, mf, nb, ps, po = _sc_rows(
        cx, cy, cz, bi, sdf_c, occ_c, t_flat.reshape(B * 16))

    fview = features.reshape(N_VIEW, 128)
    proj_features = _mask_features(
        fview, mf.reshape(N_PAD // 128, 128)).reshape(n, CH)

    historical_coords = jnp.stack([hx[:n], hy[:n], hz[:n]], axis=1)
    proj_sdf = ps[:n].reshape(n, 1)
    proj_occupancy = po[:n].reshape(n, 1)
    normalized_batch_inds = nb[:n]
    mask = mf[:n].astype(jnp.bool_)
    return (proj_features, proj_sdf, proj_occupancy, historical_coords,
            normalized_batch_inds, mask)


# speedup vs baseline: 1.0760x; 1.0760x over previous
import functools

import jax
import jax.numpy as jnp
from jax import lax
from jax.experimental import pallas as pl
from jax.experimental.pallas import tpu as pltpu
from jax.experimental.pallas import tpu_sc as plsc

N_VOX = 500000
CH = 64
B = 8
VOX = 0.0625
BX = 6.0
BY = 6.0
BZ = 3.0

NC = 2
NS = 16
NW = NC * NS
LANES = 16
CHUNK = 4000
PER_W = 16000
N_PAD = NW * PER_W
R_BLK = 16384


def _col(ref, i, j):
    return ref[:, 4 * i + j:4 * i + j + 1]


def _transform_body(inv_ref, hist_ref, out_ref):
    inv_b = inv_ref[...].astype(jnp.bfloat16).astype(jnp.float32)
    hist_b = hist_ref[...].astype(jnp.bfloat16).astype(jnp.float32)
    binv = [[inv_b[:, 4 * i + j:4 * i + j + 1] for j in range(4)]
            for i in range(4)]
    h = [[hist_b[:, 4 * i + j:4 * i + j + 1] for j in range(4)]
         for i in range(4)]
    cols = []
    for i in range(4):
        for k in range(4):
            cols.append(sum(binv[i][j] * h[j][k] for j in range(4)))
    out_ref[...] = jnp.concatenate(cols, axis=1)


def _compute_transform(inv_flat, hist_flat):
    return pl.pallas_call(
        _transform_body,
        out_shape=jax.ShapeDtypeStruct((B, 16), jnp.float32),
    )(inv_flat, hist_flat)


def _sc_body(cx_h, cy_h, cz_h, bi_h, sdf_h, occ_h, t_h,
             hx_h, hy_h, hz_h, mf_h, nb_h, ps_h, po_h,
             cx_v, cy_v, cz_v, bi_v, sdf_v, occ_v,
             hx_v, hy_v, hz_v, mf_v, nb_v, ps_v, po_v, t_v):
    wid = lax.axis_index("s") * NC + lax.axis_index("c")
    pltpu.sync_copy(t_h, t_v)
    for c in range(PER_W // CHUNK):
        base = wid * PER_W + c * CHUNK
        pltpu.sync_copy(cx_h.at[pl.ds(base, CHUNK)], cx_v)
        pltpu.sync_copy(cy_h.at[pl.ds(base, CHUNK)], cy_v)
        pltpu.sync_copy(cz_h.at[pl.ds(base, CHUNK)], cz_v)
        pltpu.sync_copy(bi_h.at[pl.ds(base, CHUNK)], bi_v)
        pltpu.sync_copy(sdf_h.at[pl.ds(base, CHUNK)], sdf_v)
        pltpu.sync_copy(occ_h.at[pl.ds(base, CHUNK)], occ_v)

        def body(i, carry):
            s = i * LANES
            bi = bi_v[pl.ds(s, LANES)]
            nb = lax.rem(bi, B)
            nb16 = nb * 16
            t = [plsc.load_gather(t_v, [nb16 + k]) for k in range(12)]
            cx = cx_v[pl.ds(s, LANES)]
            cy = cy_v[pl.ds(s, LANES)]
            cz = cz_v[pl.ds(s, LANES)]
            hx = cx * t[0] + cy * t[1] + cz * t[2] + t[3]
            hy = cx * t[4] + cy * t[5] + cz * t[6] + t[7]
            hz = cx * t[8] + cy * t[9] + cz * t[10] + t[11]
            m = ((hx >= 0.0) & (hx < BX) & (hy >= 0.0) & (hy < BY)
                 & (hz >= 0.0) & (hz < BZ))
            zero = jnp.zeros((LANES,), jnp.float32)
            hx_v[pl.ds(s, LANES)] = hx
            hy_v[pl.ds(s, LANES)] = hy
            hz_v[pl.ds(s, LANES)] = hz
            mf_v[pl.ds(s, LANES)] = jnp.where(m, 1.0, zero)
            nb_v[pl.ds(s, LANES)] = nb
            ps_v[pl.ds(s, LANES)] = jnp.where(m, sdf_v[pl.ds(s, LANES)], zero)
            po_v[pl.ds(s, LANES)] = jnp.where(m, occ_v[pl.ds(s, LANES)], zero)
            return carry

        lax.fori_loop(0, CHUNK // LANES, body, 0)
        pltpu.sync_copy(hx_v, hx_h.at[pl.ds(base, CHUNK)])
        pltpu.sync_copy(hy_v, hy_h.at[pl.ds(base, CHUNK)])
        pltpu.sync_copy(hz_v, hz_h.at[pl.ds(base, CHUNK)])
        pltpu.sync_copy(mf_v, mf_h.at[pl.ds(base, CHUNK)])
        pltpu.sync_copy(nb_v, nb_h.at[pl.ds(base, CHUNK)])
        pltpu.sync_copy(ps_v, ps_h.at[pl.ds(base, CHUNK)])
        pltpu.sync_copy(po_v, po_h.at[pl.ds(base, CHUNK)])


def _sc_rows(cx, cy, cz, bi, sdf_c, occ_c, t_flat):
    f32 = jnp.float32
    i32 = jnp.int32
    vmem_f = pltpu.VMEM((CHUNK,), f32)
    vmem_i = pltpu.VMEM((CHUNK,), i32)
    mesh = plsc.VectorSubcoreMesh(core_axis_name="c", subcore_axis_name="s")
    fn = functools.partial(
        pl.kernel,
        mesh=mesh,
        compiler_params=pltpu.CompilerParams(needs_layout_passes=False),
        out_type=[
            jax.ShapeDtypeStruct((N_PAD,), f32),
            jax.ShapeDtypeStruct((N_PAD,), f32),
            jax.ShapeDtypeStruct((N_PAD,), f32),
            jax.ShapeDtypeStruct((N_PAD,), f32),
            jax.ShapeDtypeStruct((N_PAD,), i32),
            jax.ShapeDtypeStruct((N_PAD,), f32),
            jax.ShapeDtypeStruct((N_PAD,), f32),
        ],
        scratch_types=[
            vmem_f, vmem_f, vmem_f, vmem_i, vmem_f, vmem_f,
            vmem_f, vmem_f, vmem_f, vmem_f, vmem_i, vmem_f, vmem_f,
            pltpu.VMEM((B * 16,), f32),
        ],
    )(_sc_body)
    return fn(cx, cy, cz, bi, sdf_c, occ_c, t_flat)


FR = 4096
NFULL = N_VOX // FR
TAIL = N_VOX - NFULL * FR
NBUF = 3


def _mask_col(m, rows):
    mrows = m.shape[0]
    mrep = jnp.broadcast_to(m[:, None, :], (mrows, 128, 128))
    mrep = mrep.reshape(mrows * 128, 128)[:rows]
    lane = lax.broadcasted_iota(jnp.int32, (rows, 128), 1)
    row = lax.broadcasted_iota(jnp.int32, (rows, 128), 0)
    sel = (lane == (row % 128)).astype(jnp.float32)
    return jnp.dot(mrep * sel, jnp.ones((128, 1), jnp.float32))


def _feat_body(f_hbm, m_hbm, o_hbm, fbuf, mbuf, obuf, in_sem, m_sem, out_sem):
    def start_in(i, slot):
        pltpu.make_async_copy(
            f_hbm.at[pl.ds(i * FR, FR), :], fbuf.at[slot], in_sem.at[slot]
        ).start()
        pltpu.make_async_copy(
            m_hbm.at[pl.ds(i * (FR // 128), FR // 128), :], mbuf.at[slot],
            m_sem.at[slot]
        ).start()

    for i in range(NBUF):
        start_in(i, i)

    def step(i, carry):
        slot = lax.rem(i, NBUF)
        pltpu.make_async_copy(
            f_hbm.at[pl.ds(i * FR, FR), :], fbuf.at[slot], in_sem.at[slot]
        ).wait()
        pltpu.make_async_copy(
            m_hbm.at[pl.ds(i * (FR // 128), FR // 128), :], mbuf.at[slot],
            m_sem.at[slot]
        ).wait()

        @pl.when(i >= NBUF)
        def _():
            pltpu.make_async_copy(
                obuf.at[slot], o_hbm.at[pl.ds((i - NBUF) * FR, FR), :],
                out_sem.at[slot]
            ).wait()

        mcol = _mask_col(mbuf[slot], FR)
        obuf[slot, :, :] = fbuf[slot] * mcol
        pltpu.make_async_copy(
            obuf.at[slot], o_hbm.at[pl.ds(i * FR, FR), :], out_sem.at[slot]
        ).start()

        @pl.when(i + NBUF < NFULL)
        def _():
            start_in(i + NBUF, slot)

        return carry

    lax.fori_loop(0, NFULL, step, 0)

    for k in range(NFULL - NBUF, NFULL):
        slot = k % NBUF
        pltpu.make_async_copy(
            obuf.at[slot], o_hbm.at[pl.ds(k * FR, FR), :], out_sem.at[slot]
        ).wait()

    mrows_t = (TAIL + 127) // 128
    pltpu.make_async_copy(
        f_hbm.at[pl.ds(NFULL * FR, TAIL), :], fbuf.at[0, pl.ds(0, TAIL)],
        in_sem.at[0]
    ).start()
    pltpu.make_async_copy(
        m_hbm.at[pl.ds(NFULL * (FR // 128), mrows_t), :],
        mbuf.at[0, pl.ds(0, mrows_t)], m_sem.at[0]
    ).start()
    pltpu.make_async_copy(
        f_hbm.at[pl.ds(NFULL * FR, TAIL), :], fbuf.at[0, pl.ds(0, TAIL)],
        in_sem.at[0]
    ).wait()
    pltpu.make_async_copy(
        m_hbm.at[pl.ds(NFULL * (FR // 128), mrows_t), :],
        mbuf.at[0, pl.ds(0, mrows_t)], m_sem.at[0]
    ).wait()
    mcol_t = _mask_col(mbuf[0, :mrows_t], TAIL)
    obuf[0, :TAIL, :] = fbuf[0, :TAIL] * mcol_t
    pltpu.make_async_copy(
        obuf.at[0, pl.ds(0, TAIL)], o_hbm.at[pl.ds(NFULL * FR, TAIL), :],
        out_sem.at[0]
    ).start()
    pltpu.make_async_copy(
        obuf.at[0, pl.ds(0, TAIL)], o_hbm.at[pl.ds(NFULL * FR, TAIL), :],
        out_sem.at[0]
    ).wait()


def _mask_features(features, mask_rows):
    return pl.pallas_call(
        _feat_body,
        in_specs=[
            pl.BlockSpec(memory_space=pl.ANY),
            pl.BlockSpec(memory_space=pl.ANY),
        ],
        out_specs=pl.BlockSpec(memory_space=pl.ANY),
        out_shape=jax.ShapeDtypeStruct((N_VOX, CH), jnp.float32),
        scratch_shapes=[
            pltpu.VMEM((NBUF, FR, CH), jnp.float32),
            pltpu.VMEM((NBUF, FR // 128, 128), jnp.float32),
            pltpu.VMEM((NBUF, FR, CH), jnp.float32),
            pltpu.SemaphoreType.DMA((NBUF,)),
            pltpu.SemaphoreType.DMA((NBUF,)),
            pltpu.SemaphoreType.DMA((NBUF,)),
        ],
    )(features, mask_rows)


def kernel(coords, batch_inds, features, sdf, occupancy,
           historical_pose, current_pose):
    n = coords.shape[0]
    pad = N_PAD - n

    inv_current = jnp.linalg.inv(current_pose)
    t_flat = _compute_transform(
        inv_current.reshape(B, 16), historical_pose.reshape(B, 16))

    cx = jnp.pad(coords[:, 0], (0, pad))
    cy = jnp.pad(coords[:, 1], (0, pad))
    cz = jnp.pad(coords[:, 2], (0, pad))
    bi = jnp.pad(batch_inds, (0, pad))
    sdf_c = jnp.pad(sdf[:, 0], (0, pad))
    occ_c = jnp.pad(occupancy[:, 0], (0, pad))

    hx, hy, hz, mf, nb, ps, po = _sc_rows(
        cx, cy, cz, bi, sdf_c, occ_c, t_flat.reshape(B * 16))

    proj_features = _mask_features(features, mf.reshape(N_PAD // 128, 128))

    historical_coords = jnp.stack([hx[:n], hy[:n], hz[:n]], axis=1)
    proj_sdf = ps[:n].reshape(n, 1)
    proj_occupancy = po[:n].reshape(n, 1)
    normalized_batch_inds = nb[:n]
    mask = mf[:n].astype(jnp.bool_)
    return (proj_features, proj_sdf, proj_occupancy, historical_coords,
            normalized_batch_inds, mask)


# speedup vs baseline: 1.0778x; 1.0017x over previous
import functools

import jax
import jax.numpy as jnp
from jax import lax
from jax.experimental import pallas as pl
from jax.experimental.pallas import tpu as pltpu
from jax.experimental.pallas import tpu_sc as plsc

N_VOX = 500000
CH = 64
B = 8
VOX = 0.0625
BX = 6.0
BY = 6.0
BZ = 3.0

NC = 2
NS = 16
NW = NC * NS
LANES = 16
CHUNK = 4000
PER_W = 16000
N_PAD = NW * PER_W
R_BLK = 16384


def _col(ref, i, j):
    return ref[:, 4 * i + j:4 * i + j + 1]


def _transform_body(inv_ref, hist_ref, out_ref):
    inv_b = inv_ref[...].astype(jnp.bfloat16).astype(jnp.float32)
    hist_b = hist_ref[...].astype(jnp.bfloat16).astype(jnp.float32)
    binv = [[inv_b[:, 4 * i + j:4 * i + j + 1] for j in range(4)]
            for i in range(4)]
    h = [[hist_b[:, 4 * i + j:4 * i + j + 1] for j in range(4)]
         for i in range(4)]
    cols = []
    for i in range(4):
        for k in range(4):
            cols.append(sum(binv[i][j] * h[j][k] for j in range(4)))
    out_ref[...] = jnp.concatenate(cols, axis=1)


def _compute_transform(inv_flat, hist_flat):
    return pl.pallas_call(
        _transform_body,
        out_shape=jax.ShapeDtypeStruct((B, 16), jnp.float32),
    )(inv_flat, hist_flat)


def _sc_body(cx_h, cy_h, cz_h, bi_h, sdf_h, occ_h, t_h,
             hx_h, hy_h, hz_h, mf_h, nb_h, ps_h, po_h,
             cx_v, cy_v, cz_v, bi_v, sdf_v, occ_v,
             hx_v, hy_v, hz_v, mf_v, nb_v, ps_v, po_v, t_v):
    wid = lax.axis_index("s") * NC + lax.axis_index("c")
    pltpu.sync_copy(t_h, t_v)
    for c in range(PER_W // CHUNK):
        base = wid * PER_W + c * CHUNK
        pltpu.sync_copy(cx_h.at[pl.ds(base, CHUNK)], cx_v)
        pltpu.sync_copy(cy_h.at[pl.ds(base, CHUNK)], cy_v)
        pltpu.sync_copy(cz_h.at[pl.ds(base, CHUNK)], cz_v)
        pltpu.sync_copy(bi_h.at[pl.ds(base, CHUNK)], bi_v)
        pltpu.sync_copy(sdf_h.at[pl.ds(base, CHUNK)], sdf_v)
        pltpu.sync_copy(occ_h.at[pl.ds(base, CHUNK)], occ_v)

        def body(i, carry):
            s = i * LANES
            bi = bi_v[pl.ds(s, LANES)]
            nb = lax.rem(bi, B)
            nb16 = nb * 16
            t = [plsc.load_gather(t_v, [nb16 + k]) for k in range(12)]
            cx = cx_v[pl.ds(s, LANES)]
            cy = cy_v[pl.ds(s, LANES)]
            cz = cz_v[pl.ds(s, LANES)]
            hx = cx * t[0] + cy * t[1] + cz * t[2] + t[3]
            hy = cx * t[4] + cy * t[5] + cz * t[6] + t[7]
            hz = cx * t[8] + cy * t[9] + cz * t[10] + t[11]
            m = ((hx >= 0.0) & (hx < BX) & (hy >= 0.0) & (hy < BY)
                 & (hz >= 0.0) & (hz < BZ))
            zero = jnp.zeros((LANES,), jnp.float32)
            hx_v[pl.ds(s, LANES)] = hx
            hy_v[pl.ds(s, LANES)] = hy
            hz_v[pl.ds(s, LANES)] = hz
            mf_v[pl.ds(s, LANES)] = jnp.where(m, 1.0, zero)
            nb_v[pl.ds(s, LANES)] = nb
            ps_v[pl.ds(s, LANES)] = jnp.where(m, sdf_v[pl.ds(s, LANES)], zero)
            po_v[pl.ds(s, LANES)] = jnp.where(m, occ_v[pl.ds(s, LANES)], zero)
            return carry

        lax.fori_loop(0, CHUNK // LANES, body, 0)
        pltpu.sync_copy(hx_v, hx_h.at[pl.ds(base, CHUNK)])
        pltpu.sync_copy(hy_v, hy_h.at[pl.ds(base, CHUNK)])
        pltpu.sync_copy(hz_v, hz_h.at[pl.ds(base, CHUNK)])
        pltpu.sync_copy(mf_v, mf_h.at[pl.ds(base, CHUNK)])
        pltpu.sync_copy(nb_v, nb_h.at[pl.ds(base, CHUNK)])
        pltpu.sync_copy(ps_v, ps_h.at[pl.ds(base, CHUNK)])
        pltpu.sync_copy(po_v, po_h.at[pl.ds(base, CHUNK)])


def _sc_rows(cx, cy, cz, bi, sdf_c, occ_c, t_flat):
    f32 = jnp.float32
    i32 = jnp.int32
    vmem_f = pltpu.VMEM((CHUNK,), f32)
    vmem_i = pltpu.VMEM((CHUNK,), i32)
    mesh = plsc.VectorSubcoreMesh(core_axis_name="c", subcore_axis_name="s")
    fn = functools.partial(
        pl.kernel,
        mesh=mesh,
        compiler_params=pltpu.CompilerParams(needs_layout_passes=False),
        out_type=[
            jax.ShapeDtypeStruct((N_PAD,), f32),
            jax.ShapeDtypeStruct((N_PAD,), f32),
            jax.ShapeDtypeStruct((N_PAD,), f32),
            jax.ShapeDtypeStruct((N_PAD,), f32),
            jax.ShapeDtypeStruct((N_PAD,), i32),
            jax.ShapeDtypeStruct((N_PAD,), f32),
            jax.ShapeDtypeStruct((N_PAD,), f32),
        ],
        scratch_types=[
            vmem_f, vmem_f, vmem_f, vmem_i, vmem_f, vmem_f,
            vmem_f, vmem_f, vmem_f, vmem_f, vmem_i, vmem_f, vmem_f,
            pltpu.VMEM((B * 16,), f32),
        ],
    )(_sc_body)
    return fn(cx, cy, cz, bi, sdf_c, occ_c, t_flat)


FR = 4096
NFULL = N_VOX // FR
TAIL = N_VOX - NFULL * FR
NBUF = 3


def _mask_col(m, rows):
    mrows = m.shape[0]
    mrep = jnp.broadcast_to(m[:, None, :], (mrows, 128, 128))
    mrep = mrep.reshape(mrows * 128, 128)[:rows]
    lane = lax.broadcasted_iota(jnp.int32, (rows, 128), 1)
    row = lax.broadcasted_iota(jnp.int32, (rows, 128), 0)
    sel = (lane == (row % 128)).astype(jnp.float32)
    return jnp.dot(mrep * sel, jnp.ones((128, 1), jnp.float32))


H = FR // 2


def _feat_body(f_hbm, m_hbm, o_hbm, fbuf, mbuf, obuf, in_sem, m_sem, out_sem):
    def in_copies(i, slot):
        return [pltpu.make_async_copy(
            f_hbm.at[pl.ds(i * FR + h * H, H), :],
            fbuf.at[slot, pl.ds(h * H, H)], in_sem.at[slot, h])
            for h in range(2)]

    def out_copies(i, slot):
        return [pltpu.make_async_copy(
            obuf.at[slot, pl.ds(h * H, H)],
            o_hbm.at[pl.ds(i * FR + h * H, H), :], out_sem.at[slot, h])
            for h in range(2)]

    def start_in(i, slot):
        for cp in in_copies(i, slot):
            cp.start()
        pltpu.make_async_copy(
            m_hbm.at[pl.ds(i * (FR // 128), FR // 128), :], mbuf.at[slot],
            m_sem.at[slot]
        ).start()

    for i in range(NBUF):
        start_in(i, i)

    def step(i, carry):
        slot = lax.rem(i, NBUF)
        for cp in in_copies(i, slot):
            cp.wait()
        pltpu.make_async_copy(
            m_hbm.at[pl.ds(i * (FR // 128), FR // 128), :], mbuf.at[slot],
            m_sem.at[slot]
        ).wait()

        @pl.when(i >= NBUF)
        def _():
            for cp in out_copies(i - NBUF, slot):
                cp.wait()

        mcol = _mask_col(mbuf[slot], FR)
        obuf[slot, :, :] = fbuf[slot] * mcol
        for cp in out_copies(i, slot):
            cp.start()

        @pl.when(i + NBUF < NFULL)
        def _():
            start_in(i + NBUF, slot)

        return carry

    lax.fori_loop(0, NFULL, step, 0)

    for k in range(NFULL - NBUF, NFULL):
        for cp in out_copies(k, k % NBUF):
            cp.wait()

    mrows_t = (TAIL + 127) // 128
    pltpu.make_async_copy(
        f_hbm.at[pl.ds(NFULL * FR, TAIL), :], fbuf.at[0, pl.ds(0, TAIL)],
        in_sem.at[0, 0]
    ).start()
    pltpu.make_async_copy(
        m_hbm.at[pl.ds(NFULL * (FR // 128), mrows_t), :],
        mbuf.at[0, pl.ds(0, mrows_t)], m_sem.at[0]
    ).start()
    pltpu.make_async_copy(
        f_hbm.at[pl.ds(NFULL * FR, TAIL), :], fbuf.at[0, pl.ds(0, TAIL)],
        in_sem.at[0, 0]
    ).wait()
    pltpu.make_async_copy(
        m_hbm.at[pl.ds(NFULL * (FR // 128), mrows_t), :],
        mbuf.at[0, pl.ds(0, mrows_t)], m_sem.at[0]
    ).wait()
    mcol_t = _mask_col(mbuf[0, :mrows_t], TAIL)
    obuf[0, :TAIL, :] = fbuf[0, :TAIL] * mcol_t
    pltpu.make_async_copy(
        obuf.at[0, pl.ds(0, TAIL)], o_hbm.at[pl.ds(NFULL * FR, TAIL), :],
        out_sem.at[0, 0]
    ).start()
    pltpu.make_async_copy(
        obuf.at[0, pl.ds(0, TAIL)], o_hbm.at[pl.ds(NFULL * FR, TAIL), :],
        out_sem.at[0, 0]
    ).wait()


def _mask_features(features, mask_rows):
    return pl.pallas_call(
        _feat_body,
        in_specs=[
            pl.BlockSpec(memory_space=pl.ANY),
            pl.BlockSpec(memory_space=pl.ANY),
        ],
        out_specs=pl.BlockSpec(memory_space=pl.ANY),
        out_shape=jax.ShapeDtypeStruct((N_VOX, CH), jnp.float32),
        scratch_shapes=[
            pltpu.VMEM((NBUF, FR, CH), jnp.float32),
            pltpu.VMEM((NBUF, FR // 128, 128), jnp.float32),
            pltpu.VMEM((NBUF, FR, CH), jnp.float32),
            pltpu.SemaphoreType.DMA((NBUF, 2)),
            pltpu.SemaphoreType.DMA((NBUF,)),
            pltpu.SemaphoreType.DMA((NBUF, 2)),
        ],
    )(features, mask_rows)


def kernel(coords, batch_inds, features, sdf, occupancy,
           historical_pose, current_pose):
    n = coords.shape[0]
    pad = N_PAD - n

    inv_current = jnp.linalg.inv(current_pose)
    t_flat = _compute_transform(
        inv_current.reshape(B, 16), historical_pose.reshape(B, 16))

    cx = jnp.pad(coords[:, 0], (0, pad))
    cy = jnp.pad(coords[:, 1], (0, pad))
    cz = jnp.pad(coords[:, 2], (0, pad))
    bi = jnp.pad(batch_inds, (0, pad))
    sdf_c = jnp.pad(sdf[:, 0], (0, pad))
    occ_c = jnp.pad(occupancy[:, 0], (0, pad))

    hx, hy, hz, mf, nb, ps, po = _sc_rows(
        cx, cy, cz, bi, sdf_c, occ_c, t_flat.reshape(B * 16))

    proj_features = _mask_features(features, mf.reshape(N_PAD // 128, 128))

    historical_coords = jnp.stack([hx[:n], hy[:n], hz[:n]], axis=1)
    proj_sdf = ps[:n].reshape(n, 1)
    proj_occupancy = po[:n].reshape(n, 1)
    normalized_batch_inds = nb[:n]
    mask = mf[:n].astype(jnp.bool_)
    return (proj_features, proj_sdf, proj_occupancy, historical_coords,
            normalized_batch_inds, mask)


# speedup vs baseline: 1.1618x; 1.0780x over previous
import functools

import jax
import jax.numpy as jnp
from jax import lax
from jax.experimental import pallas as pl
from jax.experimental.pallas import tpu as pltpu
from jax.experimental.pallas import tpu_sc as plsc

N_VOX = 500000
CH = 64
B = 8
VOX = 0.0625
BX = 6.0
BY = 6.0
BZ = 3.0

NC = 2
NS = 16
NW = NC * NS
LANES = 16
CHUNK = 8000
PER_W = 16000
N_PAD = NW * PER_W
R_BLK = 16384


def _col(ref, i, j):
    return ref[:, 4 * i + j:4 * i + j + 1]


def _transform_body(inv_ref, hist_ref, out_ref):
    inv_b = inv_ref[...].astype(jnp.bfloat16).astype(jnp.float32)
    hist_b = hist_ref[...].astype(jnp.bfloat16).astype(jnp.float32)
    binv = [[inv_b[:, 4 * i + j:4 * i + j + 1] for j in range(4)]
            for i in range(4)]
    h = [[hist_b[:, 4 * i + j:4 * i + j + 1] for j in range(4)]
         for i in range(4)]
    cols = []
    for i in range(4):
        for k in range(4):
            cols.append(sum(binv[i][j] * h[j][k] for j in range(4)))
    out_ref[...] = jnp.concatenate(cols, axis=1)


def _compute_transform(inv_flat, hist_flat):
    return pl.pallas_call(
        _transform_body,
        out_shape=jax.ShapeDtypeStruct((B, 16), jnp.float32),
    )(inv_flat, hist_flat)


def _sc_body(cx_h, cy_h, cz_h, bi_h, sdf_h, occ_h, t_h,
             hx_h, hy_h, hz_h, mf_h, nb_h, ps_h, po_h,
             cx_v, cy_v, cz_v, bi_v, sdf_v, occ_v,
             hx_v, hy_v, hz_v, mf_v, nb_v, ps_v, po_v, t_v):
    wid = lax.axis_index("s") * NC + lax.axis_index("c")
    pltpu.sync_copy(t_h, t_v)
    for c in range(PER_W // CHUNK):
        base = wid * PER_W + c * CHUNK
        pltpu.sync_copy(cx_h.at[pl.ds(base, CHUNK)], cx_v)
        pltpu.sync_copy(cy_h.at[pl.ds(base, CHUNK)], cy_v)
        pltpu.sync_copy(cz_h.at[pl.ds(base, CHUNK)], cz_v)
        pltpu.sync_copy(bi_h.at[pl.ds(base, CHUNK)], bi_v)
        pltpu.sync_copy(sdf_h.at[pl.ds(base, CHUNK)], sdf_v)
        pltpu.sync_copy(occ_h.at[pl.ds(base, CHUNK)], occ_v)

        def body(i, carry):
            s = i * LANES
            bi = bi_v[pl.ds(s, LANES)]
            nb = lax.rem(bi, B)
            nb16 = nb * 16
            t = [plsc.load_gather(t_v, [nb16 + k]) for k in range(12)]
            cx = cx_v[pl.ds(s, LANES)]
            cy = cy_v[pl.ds(s, LANES)]
            cz = cz_v[pl.ds(s, LANES)]
            hx = cx * t[0] + cy * t[1] + cz * t[2] + t[3]
            hy = cx * t[4] + cy * t[5] + cz * t[6] + t[7]
            hz = cx * t[8] + cy * t[9] + cz * t[10] + t[11]
            m = ((hx >= 0.0) & (hx < BX) & (hy >= 0.0) & (hy < BY)
                 & (hz >= 0.0) & (hz < BZ))
            zero = jnp.zeros((LANES,), jnp.float32)
            hx_v[pl.ds(s, LANES)] = hx
            hy_v[pl.ds(s, LANES)] = hy
            hz_v[pl.ds(s, LANES)] = hz
            mf_v[pl.ds(s, LANES)] = jnp.where(m, 1.0, zero)
            nb_v[pl.ds(s, LANES)] = nb
            ps_v[pl.ds(s, LANES)] = jnp.where(m, sdf_v[pl.ds(s, LANES)], zero)
            po_v[pl.ds(s, LANES)] = jnp.where(m, occ_v[pl.ds(s, LANES)], zero)
            return carry

        lax.fori_loop(0, CHUNK // LANES, body, 0)
        pltpu.sync_copy(hx_v, hx_h.at[pl.ds(base, CHUNK)])
        pltpu.sync_copy(hy_v, hy_h.at[pl.ds(base, CHUNK)])
        pltpu.sync_copy(hz_v, hz_h.at[pl.ds(base, CHUNK)])
        pltpu.sync_copy(mf_v, mf_h.at[pl.ds(base, CHUNK)])
        pltpu.sync_copy(nb_v, nb_h.at[pl.ds(base, CHUNK)])
        pltpu.sync_copy(ps_v, ps_h.at[pl.ds(base, CHUNK)])
        pltpu.sync_copy(po_v, po_h.at[pl.ds(base, CHUNK)])


def _sc_rows(cx, cy, cz, bi, sdf_c, occ_c, t_flat):
    f32 = jnp.float32
    i32 = jnp.int32
    vmem_f = pltpu.VMEM((CHUNK,), f32)
    vmem_i = pltpu.VMEM((CHUNK,), i32)
    mesh = plsc.VectorSubcoreMesh(core_axis_name="c", subcore_axis_name="s")
    fn = functools.partial(
        pl.kernel,
        mesh=mesh,
        compiler_params=pltpu.CompilerParams(needs_layout_passes=False),
        out_type=[
            jax.ShapeDtypeStruct((N_PAD,), f32),
            jax.ShapeDtypeStruct((N_PAD,), f32),
            jax.ShapeDtypeStruct((N_PAD,), f32),
            jax.ShapeDtypeStruct((N_PAD,), f32),
            jax.ShapeDtypeStruct((N_PAD,), i32),
            jax.ShapeDtypeStruct((N_PAD,), f32),
            jax.ShapeDtypeStruct((N_PAD,), f32),
        ],
        scratch_types=[
            vmem_f, vmem_f, vmem_f, vmem_i, vmem_f, vmem_f,
            vmem_f, vmem_f, vmem_f, vmem_f, vmem_i, vmem_f, vmem_f,
            pltpu.VMEM((B * 16,), f32),
        ],
    )(_sc_body)
    return fn(cx, cy, cz, bi, sdf_c, occ_c, t_flat)


FR = 4096
NFULL = N_VOX // FR
TAIL = N_VOX - NFULL * FR
NBUF = 3


def _mask_col(m, rows):
    mrows = m.shape[0]
    mrep = jnp.broadcast_to(m[:, None, :], (mrows, 128, 128))
    mrep = mrep.reshape(mrows * 128, 128)[:rows]
    lane = lax.broadcasted_iota(jnp.int32, (rows, 128), 1)
    row = lax.broadcasted_iota(jnp.int32, (rows, 128), 0)
    sel = (lane == (row % 128)).astype(jnp.float32)
    return jnp.dot(mrep * sel, jnp.ones((128, 1), jnp.float32))


H = FR // 2


def _feat_body(f_hbm, m_hbm, o_hbm, fbuf, mbuf, obuf, in_sem, m_sem, out_sem):
    def in_copies(i, slot):
        return [pltpu.make_async_copy(
            f_hbm.at[pl.ds(i * FR + h * H, H), :],
            fbuf.at[slot, pl.ds(h * H, H)], in_sem.at[slot, h])
            for h in range(2)]

    def out_copies(i, slot):
        return [pltpu.make_async_copy(
            obuf.at[slot, pl.ds(h * H, H)],
            o_hbm.at[pl.ds(i * FR + h * H, H), :], out_sem.at[slot, h])
            for h in range(2)]

    def start_in(i, slot):
        for cp in in_copies(i, slot):
            cp.start()
        pltpu.make_async_copy(
            m_hbm.at[pl.ds(i * (FR // 128), FR // 128), :], mbuf.at[slot],
            m_sem.at[slot]
        ).start()

    for i in range(NBUF):
        start_in(i, i)

    def step(i, carry):
        slot = lax.rem(i, NBUF)
        for cp in in_copies(i, slot):
            cp.wait()
        pltpu.make_async_copy(
            m_hbm.at[pl.ds(i * (FR // 128), FR // 128), :], mbuf.at[slot],
            m_sem.at[slot]
        ).wait()

        @pl.when(i >= NBUF)
        def _():
            for cp in out_copies(i - NBUF, slot):
                cp.wait()

        mcol = _mask_col(mbuf[slot], FR)
        obuf[slot, :, :] = fbuf[slot] * mcol
        for cp in out_copies(i, slot):
            cp.start()

        @pl.when(i + NBUF < NFULL)
        def _():
            start_in(i + NBUF, slot)

        return carry

    lax.fori_loop(0, NFULL, step, 0)

    for k in range(NFULL - NBUF, NFULL):
        for cp in out_copies(k, k % NBUF):
            cp.wait()

    mrows_t = (TAIL + 127) // 128
    pltpu.make_async_copy(
        f_hbm.at[pl.ds(NFULL * FR, TAIL), :], fbuf.at[0, pl.ds(0, TAIL)],
        in_sem.at[0, 0]
    ).start()
    pltpu.make_async_copy(
        m_hbm.at[pl.ds(NFULL * (FR // 128), mrows_t), :],
        mbuf.at[0, pl.ds(0, mrows_t)], m_sem.at[0]
    ).start()
    pltpu.make_async_copy(
        f_hbm.at[pl.ds(NFULL * FR, TAIL), :], fbuf.at[0, pl.ds(0, TAIL)],
        in_sem.at[0, 0]
    ).wait()
    pltpu.make_async_copy(
        m_hbm.at[pl.ds(NFULL * (FR // 128), mrows_t), :],
        mbuf.at[0, pl.ds(0, mrows_t)], m_sem.at[0]
    ).wait()
    mcol_t = _mask_col(mbuf[0, :mrows_t], TAIL)
    obuf[0, :TAIL, :] = fbuf[0, :TAIL] * mcol_t
    pltpu.make_async_copy(
        obuf.at[0, pl.ds(0, TAIL)], o_hbm.at[pl.ds(NFULL * FR, TAIL), :],
        out_sem.at[0, 0]
    ).start()
    pltpu.make_async_copy(
        obuf.at[0, pl.ds(0, TAIL)], o_hbm.at[pl.ds(NFULL * FR, TAIL), :],
        out_sem.at[0, 0]
    ).wait()


def _mask_features(features, mask_rows):
    return pl.pallas_call(
        _feat_body,
        in_specs=[
            pl.BlockSpec(memory_space=pl.ANY),
            pl.BlockSpec(memory_space=pl.ANY),
        ],
        out_specs=pl.BlockSpec(memory_space=pl.ANY),
        out_shape=jax.ShapeDtypeStruct((N_VOX, CH), jnp.float32),
        scratch_shapes=[
            pltpu.VMEM((NBUF, FR, CH), jnp.float32),
            pltpu.VMEM((NBUF, FR // 128, 128), jnp.float32),
            pltpu.VMEM((NBUF, FR, CH), jnp.float32),
            pltpu.SemaphoreType.DMA((NBUF, 2)),
            pltpu.SemaphoreType.DMA((NBUF,)),
            pltpu.SemaphoreType.DMA((NBUF, 2)),
        ],
    )(features, mask_rows)


def kernel(coords, batch_inds, features, sdf, occupancy,
           historical_pose, current_pose):
    n = coords.shape[0]
    pad = N_PAD - n

    inv_current = jnp.linalg.inv(current_pose)
    t_flat = _compute_transform(
        inv_current.reshape(B, 16), historical_pose.reshape(B, 16))

    coords_p = jnp.pad(coords, ((0, pad), (0, 0)))
    cx = coords_p[:, 0]
    cy = coords_p[:, 1]
    cz = coords_p[:, 2]
    bi = jnp.pad(batch_inds, (0, pad))
    sdf_c = jnp.pad(sdf, ((0, pad), (0, 0))).reshape(N_PAD)
    occ_c = jnp.pad(occupancy, ((0, pad), (0, 0))).reshape(N_PAD)

    hx, hy, hz, mf, nb, ps, po = _sc_rows(
        cx, cy, cz, bi, sdf_c, occ_c, t_flat.reshape(B * 16))

    proj_features = _mask_features(features, mf.reshape(N_PAD // 128, 128))

    historical_coords = jnp.stack([hx[:n], hy[:n], hz[:n]], axis=1)
    proj_sdf = ps[:n].reshape(n, 1)
    proj_occupancy = po[:n].reshape(n, 1)
    normalized_batch_inds = nb[:n]
    mask = mf[:n].astype(jnp.bool_)
    return (proj_features, proj_sdf, proj_occupancy, historical_coords,
            normalized_batch_inds, mask)
